# baseline (speedup 1.0000x reference)
import jax
import jax.numpy as jnp
from jax import lax
from jax.experimental import pallas as pl
from jax.experimental.pallas import tpu as pltpu
from jax.experimental.pallas import tpu_sc as plsc

U_NUM = 25000
I_NUM = 25000
N = U_NUM + I_NUM
E = 800000
D = 64
HALF = 32
N_LAYERS = 3

NC = 2
NS = 16
CHUNK = 272
CHUNKS_PER_TILE = 184
E_PAD = CHUNKS_PER_TILE * CHUNK * NS
N_PAD = 50048
ROWS_PER_TILE = N_PAD // NS
PCONV = 136
NIDX = 4


def _sfa_body(ego0, pack, zeros, l1, l2, l3,
              i0, i1, i2, i3, rows0, rows1, scat0, scat1,
              acc, is0, is1, is2, is3, gsem0, gsem1, ssem0, ssem1):
    idxb = (i0, i1, i2, i3)
    isem = (is0, is1, is2, is3)
    rows = (rows0, rows1)
    scat = (scat0, scat1)
    gsem = (gsem0, gsem1)
    ssem = (ssem0, ssem1)
    c = lax.axis_index("c")
    t = lax.axis_index("s")
    r0 = t * ROWS_PER_TILE
    K = CHUNKS_PER_TILE

    layer_bufs = [ego0, l1, l2, l3]
    for layer in range(N_LAYERS):
        cur = layer_bufs[layer]
        nxt = layer_bufs[layer + 1]

        def fire_idx(j, r):
            pltpu.async_copy(pack.at[t * K + j], idxb[r], isem[r])

        def wait_idx(j, r):
            pltpu.make_async_copy(pack.at[t * K + j], idxb[r], isem[r]).wait()

        def fire_gathers(j, b, r, cur=cur):
            pltpu.async_copy(cur.at[c].at[idxb[r].at[pl.ds(0, CHUNK)]],
                             rows[b], gsem[b])

        def wait_gathers(b, r, cur=cur):
            pltpu.make_async_copy(cur.at[c].at[idxb[r].at[pl.ds(0, CHUNK)]],
                                  rows[b], gsem[b]).wait()

        def multiply(b, r):
            rows_v = rows[b]
            scat_v = scat[b]

            def mul_body(g, _):
                wv = plsc.bitcast(
                    idxb[r][pl.ds(2 * CHUNK + g * 16, 16)], jnp.float32)
                e = g * 16
                for i in range(16):
                    ws = jnp.take_along_axis(
                        wv, jnp.full((16,), i, jnp.int32), axis=0)
                    fa, fb = plsc.unpack(rows_v[e + i, pl.ds(0, 32)],
                                         format=plsc.PackFormat.INTERLEAVED)
                    scat_v[e + i, pl.ds(0, 16)] = fa * ws
                    scat_v[e + i, pl.ds(16, 16)] = fb * ws
                return 0
            lax.fori_loop(0, CHUNK // 16, mul_body, 0)

        def fire_scatter(b, r):
            pltpu.async_copy(scat[b],
                             acc.at[idxb[r].at[pl.ds(CHUNK, CHUNK)]], ssem[b],
                             add=True)

        def wait_scatter(b, r):
            pltpu.make_async_copy(scat[b],
                                  acc.at[idxb[r].at[pl.ds(CHUNK, CHUNK)]],
                                  ssem[b]).wait()

        pltpu.sync_copy(zeros.at[pl.ds(r0, ROWS_PER_TILE)],
                        acc.at[pl.ds(r0, ROWS_PER_TILE)])
        plsc.subcore_barrier()

        fire_idx(0, 0)
        fire_idx(1, 1)
        fire_idx(2, 2)
        wait_idx(0, 0)
        fire_gathers(0, 0, 0)
        fire_idx(3, 3)
        wait_idx(1, 1)
        fire_gathers(1, 1, 1)
        wait_gathers(0, 0)
        multiply(0, 0)
        fire_scatter(0, 0)

        @pl.loop(1, K - 3, step=4)
        def _(k):
            for b01 in range(4):
                j = k + b01
                sl = (1 + b01) % NIDX
                b = (1 + b01) % 2
                wait_scatter(1 - b, (sl - 1) % NIDX)
                fire_idx(j + 3, (sl + 3) % NIDX)
                wait_idx(j + 1, (sl + 1) % NIDX)
                fire_gathers(j + 1, 1 - b, (sl + 1) % NIDX)
                wait_gathers(b, sl)
                multiply(b, sl)
                fire_scatter(b, sl)

        for j in (K - 3, K - 2, K - 1):
            sl = j % NIDX
            b = j % 2
            wait_scatter(1 - b, (sl - 1) % NIDX)
            if j + 1 < K:
                wait_idx(j + 1, (sl + 1) % NIDX)
                fire_gathers(j + 1, 1 - b, (sl + 1) % NIDX)
            wait_gathers(b, sl)
            multiply(b, sl)
            fire_scatter(b, sl)
        wait_scatter((K - 1) % 2, (K - 1) % NIDX)
        plsc.subcore_barrier()

        for p in range(ROWS_PER_TILE // PCONV):
            rr = r0 + p * PCONV
            pltpu.sync_copy(acc.at[pl.ds(rr, PCONV)], scat0.at[pl.ds(0, PCONV)])

            def conv_body(q, _):
                packed = plsc.pack(scat0[q, pl.ds(0, 16)],
                                   scat0[q, pl.ds(16, 16)],
                                   format=plsc.PackFormat.INTERLEAVED)
                rows0[q, pl.ds(0, 32)] = packed
                return 0
            lax.fori_loop(0, PCONV, conv_body, 0)
            pltpu.sync_copy(rows0.at[pl.ds(0, PCONV)],
                            nxt.at[c].at[pl.ds(rr, PCONV)])


_X = N_PAD * HALF // 128
_BRX = 544


def _tc_mean_body(a, b, c, d, o):
    o[...] = (a[...] + b[...].astype(jnp.float32) + c[...].astype(jnp.float32)
              + d[...].astype(jnp.float32)) * 0.25


def _tc_mean(a, b, c, d):
    spec = pl.BlockSpec((1, _BRX, 128), lambda h, r: (h, r, 0))
    return pl.pallas_call(
        _tc_mean_body,
        grid=(NC, _X // _BRX),
        in_specs=[spec] * 4,
        out_specs=spec,
        out_shape=jax.ShapeDtypeStruct((NC, _X, 128), jnp.float32),
    )(a, b, c, d)


@jax.jit
def _sfa_encoder(user_emb, item_emb, edge_index, edge_values):
    ego0 = jnp.concatenate(
        [user_emb, item_emb, jnp.zeros((N_PAD - N, D), jnp.float32)], axis=0)
    ego0_h = ego0.reshape(N_PAD, NC, HALF).transpose(1, 0, 2)
    pad = E_PAD - E
    src_p = jnp.concatenate([edge_index[0], jnp.zeros((pad,), jnp.int32)])
    dst_p = jnp.concatenate([edge_index[1], jnp.zeros((pad,), jnp.int32)])
    w_p = jnp.concatenate([edge_values, jnp.zeros((pad,), jnp.float32)])
    w_bits = lax.bitcast_convert_type(w_p, jnp.int32)
    pack = jnp.concatenate(
        [src_p.reshape(-1, CHUNK), dst_p.reshape(-1, CHUNK),
         w_bits.reshape(-1, CHUNK)], axis=1)
    zeros = jnp.zeros((N_PAD, HALF), jnp.float32)
    ego0_bf = ego0_h.astype(jnp.bfloat16)

    mesh = plsc.VectorSubcoreMesh(core_axis_name="c", subcore_axis_name="s")
    f32 = jnp.float32
    out_type = tuple(jax.ShapeDtypeStruct((NC, N_PAD, HALF), jnp.bfloat16)
                     for _ in range(3))
    kern = pl.kernel(
        _sfa_body,
        out_type=out_type,
        mesh=mesh,
        scratch_types=[pltpu.VMEM((3 * CHUNK,), jnp.int32)] * NIDX + [
            pltpu.VMEM((CHUNK, HALF), jnp.bfloat16),
            pltpu.VMEM((CHUNK, HALF), jnp.bfloat16),
            pltpu.VMEM((CHUNK, HALF), f32),
            pltpu.VMEM((CHUNK, HALF), f32),
            pltpu.VMEM_SHARED((N_PAD, HALF), f32),
        ] + [pltpu.SemaphoreType.DMA] * (NIDX + 4),
        compiler_params=pltpu.CompilerParams(use_tc_tiling_on_sc=False,
                                             needs_layout_passes=False),
    )
    l1, l2, l3 = kern(ego0_bf, pack, zeros)
    flat = lambda x: x.reshape(NC, -1, 128)
    mean_h = _tc_mean(flat(ego0_h), flat(l1), flat(l2), flat(l3))
    mean = mean_h.reshape(NC, N_PAD, HALF).transpose(1, 0, 2).reshape(N_PAD, D)
    return mean[:U_NUM], mean[U_NUM:N]


def kernel(user_emb, item_emb, edge_index, edge_values):
    return _sfa_encoder(user_emb, item_emb, edge_index, edge_values)


# speedup vs baseline: 1.3828x; 1.3828x over previous
import jax
import jax.numpy as jnp
from jax import lax
from jax.experimental import pallas as pl
from jax.experimental.pallas import tpu as pltpu
from jax.experimental.pallas import tpu_sc as plsc

U_NUM = 25000
I_NUM = 25000
N = U_NUM + I_NUM
E = 800000
D = 64
HALF = 32
N_LAYERS = 3

NC = 2
NS = 16
CHUNK = 384
CHUNKS_PER_TILE = 132
E_PAD = CHUNKS_PER_TILE * CHUNK * NS
N_PAD = 50048
ROWS_PER_TILE = N_PAD // NS
NIDX = 4


def _sfa_body(ego0, pack, zeros, l1, l2, l3,
              i0, i1, i2, i3, rows0, rows1,
              acc, is0, is1, is2, is3, gsem0, gsem1, ssem0, ssem1):
    idxb = (i0, i1, i2, i3)
    isem = (is0, is1, is2, is3)
    rows = (rows0, rows1)
    gsem = (gsem0, gsem1)
    ssem = (ssem0, ssem1)
    c = lax.axis_index("c")
    t = lax.axis_index("s")
    r0 = t * ROWS_PER_TILE
    K = CHUNKS_PER_TILE

    layer_bufs = [ego0, l1, l2, l3]
    for layer in range(N_LAYERS):
        cur = layer_bufs[layer]
        nxt = layer_bufs[layer + 1]

        def fire_idx(j, r):
            pltpu.async_copy(pack.at[t * K + j], idxb[r], isem[r])

        def wait_idx(j, r):
            pltpu.make_async_copy(pack.at[t * K + j], idxb[r], isem[r]).wait()

        def fire_gathers(j, b, r, cur=cur):
            pltpu.async_copy(cur.at[c].at[idxb[r].at[pl.ds(0, CHUNK)]],
                             rows[b], gsem[b])

        def wait_gathers(b, r, cur=cur):
            pltpu.make_async_copy(cur.at[c].at[idxb[r].at[pl.ds(0, CHUNK)]],
                                  rows[b], gsem[b]).wait()

        def multiply(b, r):
            rows_v = rows[b]

            def mul_body(g, _):
                wv = plsc.bitcast(
                    idxb[r][pl.ds(2 * CHUNK + g * 16, 16)], jnp.float32)
                e = g * 16
                for i in range(16):
                    ws = jnp.take_along_axis(
                        wv, jnp.full((16,), i, jnp.int32), axis=0)
                    rows_v[e + i, pl.ds(0, 16)] = rows_v[e + i, pl.ds(0, 16)] * ws
                    rows_v[e + i, pl.ds(16, 16)] = rows_v[e + i, pl.ds(16, 16)] * ws
                return 0
            lax.fori_loop(0, CHUNK // 16, mul_body, 0)

        def fire_scatter(b, r):
            pltpu.async_copy(rows[b],
                             acc.at[idxb[r].at[pl.ds(CHUNK, CHUNK)]], ssem[b],
                             add=True)

        def wait_scatter(b, r):
            pltpu.make_async_copy(rows[b],
                                  acc.at[idxb[r].at[pl.ds(CHUNK, CHUNK)]],
                                  ssem[b]).wait()

        pltpu.sync_copy(zeros.at[pl.ds(r0, ROWS_PER_TILE)],
                        acc.at[pl.ds(r0, ROWS_PER_TILE)])
        plsc.subcore_barrier()

        fire_idx(0, 0)
        fire_idx(1, 1)
        fire_idx(2, 2)
        wait_idx(0, 0)
        fire_gathers(0, 0, 0)
        fire_idx(3, 3)
        wait_idx(1, 1)
        fire_gathers(1, 1, 1)
        wait_gathers(0, 0)
        multiply(0, 0)
        fire_scatter(0, 0)

        @pl.loop(1, K - 3, step=4)
        def _(k):
            for b01 in range(4):
                j = k + b01
                sl = (1 + b01) % NIDX
                b = (1 + b01) % 2
                wait_scatter(1 - b, (sl - 1) % NIDX)
                fire_idx(j + 3, (sl + 3) % NIDX)
                wait_idx(j + 1, (sl + 1) % NIDX)
                fire_gathers(j + 1, 1 - b, (sl + 1) % NIDX)
                wait_gathers(b, sl)
                multiply(b, sl)
                fire_scatter(b, sl)

        for j in (K - 3, K - 2, K - 1):
            sl = j % NIDX
            b = j % 2
            wait_scatter(1 - b, (sl - 1) % NIDX)
            if j + 1 < K:
                wait_idx(j + 1, (sl + 1) % NIDX)
                fire_gathers(j + 1, 1 - b, (sl + 1) % NIDX)
            wait_gathers(b, sl)
            multiply(b, sl)
            fire_scatter(b, sl)
        wait_scatter((K - 1) % 2, (K - 1) % NIDX)
        plsc.subcore_barrier()

        pltpu.sync_copy(acc.at[pl.ds(r0, ROWS_PER_TILE)],
                        nxt.at[c].at[pl.ds(r0, ROWS_PER_TILE)])


_X = N_PAD * HALF // 128
_BRX = 544


def _tc_mean_body(a, b, c, d, o):
    o[...] = (a[...] + b[...] + c[...] + d[...]) * 0.25


def _tc_mean(a, b, c, d):
    spec = pl.BlockSpec((1, _BRX, 128), lambda h, r: (h, r, 0))
    return pl.pallas_call(
        _tc_mean_body,
        grid=(NC, _X // _BRX),
        in_specs=[spec] * 4,
        out_specs=spec,
        out_shape=jax.ShapeDtypeStruct((NC, _X, 128), jnp.float32),
    )(a, b, c, d)


@jax.jit
def _sfa_encoder(user_emb, item_emb, edge_index, edge_values):
    ego0 = jnp.concatenate(
        [user_emb, item_emb, jnp.zeros((N_PAD - N, D), jnp.float32)], axis=0)
    ego0_h = ego0.reshape(N_PAD, NC, HALF).transpose(1, 0, 2)
    pad = E_PAD - E
    src_p = jnp.concatenate([edge_index[0], jnp.zeros((pad,), jnp.int32)])
    dst_p = jnp.concatenate([edge_index[1], jnp.zeros((pad,), jnp.int32)])
    w_p = jnp.concatenate([edge_values, jnp.zeros((pad,), jnp.float32)])
    w_bits = lax.bitcast_convert_type(w_p, jnp.int32)
    pack = jnp.concatenate(
        [src_p.reshape(-1, CHUNK), dst_p.reshape(-1, CHUNK),
         w_bits.reshape(-1, CHUNK)], axis=1)
    zeros = jnp.zeros((N_PAD, HALF), jnp.float32)

    mesh = plsc.VectorSubcoreMesh(core_axis_name="c", subcore_axis_name="s")
    f32 = jnp.float32
    out_type = tuple(jax.ShapeDtypeStruct((NC, N_PAD, HALF), f32) for _ in range(3))
    kern = pl.kernel(
        _sfa_body,
        out_type=out_type,
        mesh=mesh,
        scratch_types=[pltpu.VMEM((3 * CHUNK,), jnp.int32)] * NIDX + [
            pltpu.VMEM((CHUNK, HALF), f32),
            pltpu.VMEM((CHUNK, HALF), f32),
            pltpu.VMEM_SHARED((N_PAD, HALF), f32),
        ] + [pltpu.SemaphoreType.DMA] * (NIDX + 4),
        compiler_params=pltpu.CompilerParams(use_tc_tiling_on_sc=False,
                                             needs_layout_passes=False),
    )
    l1, l2, l3 = kern(ego0_h, pack, zeros)
    flat = lambda x: x.reshape(NC, -1, 128)
    mean_h = _tc_mean(flat(ego0_h), flat(l1), flat(l2), flat(l3))
    mean = mean_h.reshape(NC, N_PAD, HALF).transpose(1, 0, 2).reshape(N_PAD, D)
    return mean[:U_NUM], mean[U_NUM:N]


def kernel(user_emb, item_emb, edge_index, edge_values):
    return _sfa_encoder(user_emb, item_emb, edge_index, edge_values)
